# baseline (speedup 1.0000x reference)
import functools

import jax
import jax.numpy as jnp
from jax import lax
from jax.experimental import pallas as pl
from jax.experimental.pallas import tpu as pltpu
from jax.experimental.pallas import tpu_sc as plsc


def _router_body(x_ref, gw_ref, idx_ref, w_ref):
    x = x_ref[...]
    gw = gw_ref[...]
    logits = lax.dot_general(x, gw, (((1,), (1,)), ((), ())),
                             preferred_element_type=jnp.float32)
    m = jnp.max(logits, axis=1, keepdims=True)
    ex = jnp.exp(logits - m)
    probs = ex / jnp.sum(ex, axis=1, keepdims=True)
    col = lax.broadcasted_iota(jnp.int32, probs.shape, 1)
    big = jnp.int32(2 ** 30)
    p1 = jnp.max(probs, axis=1, keepdims=True)
    i1 = jnp.min(jnp.where(probs == p1, col, big), axis=1, keepdims=True)
    mask1 = col == i1
    probs2 = jnp.where(mask1, -1.0, probs)
    p2 = jnp.max(probs2, axis=1, keepdims=True)
    i2 = jnp.min(jnp.where(probs2 == p2, col, big), axis=1, keepdims=True)
    idx_ref[...] = jnp.concatenate([i1, i2], axis=1)
    w_ref[...] = jnp.concatenate([p1, p2], axis=1)


def _swiglu(x, w1, w2):
    h1 = lax.dot_general(x, w1, (((1,), (1,)), ((), ())),
                         preferred_element_type=jnp.float32)
    h2 = lax.dot_general(x, w2, (((1,), (1,)), ((), ())),
                         preferred_element_type=jnp.float32)
    return (h1 * lax.logistic(h1) * h2).astype(jnp.bfloat16)


def _make_up_body(bm):
    def body(e_sm, m_sm, lo_sm, hi_sm, xg_ref, w1_ref, w2_ref, h_ref):
        g = pl.program_id(1)
        xv = xg_ref[...].astype(jnp.bfloat16)
        w1 = w1_ref[0, 0].astype(jnp.bfloat16)
        w2 = w2_ref[0, 0].astype(jnp.bfloat16)
        h1 = lax.dot_general(xv, w1, (((1,), (1,)), ((), ())),
                             preferred_element_type=jnp.float32)
        h2 = lax.dot_general(xv, w2, (((1,), (1,)), ((), ())),
                             preferred_element_type=jnp.float32)
        hv = (h1 * lax.logistic(h1) * h2).astype(jnp.bfloat16)
        m = m_sm[g]
        lo = lo_sm[g]
        hi = hi_sm[g]
        rowid = m * bm + lax.broadcasted_iota(jnp.int32, (bm, 1), 0)
        mask = (rowid >= lo) & (rowid < hi)
        h_ref[0] = jnp.where(mask, hv, h_ref[0])

    return body


def _make_down_body(bm):
    def body(e_sm, m_sm, lo_sm, hi_sm, h0_ref, h1_ref, wgt_ref, wp_ref,
             y_ref):
        g = pl.program_id(0)
        hcat = jnp.concatenate([h0_ref[0], h1_ref[0]], axis=1)
        wp = wp_ref[0].astype(jnp.bfloat16)
        out = lax.dot_general(hcat, wp, (((1,), (1,)), ((), ())),
                              preferred_element_type=jnp.float32)
        m = m_sm[g]
        lo = lo_sm[g]
        hi = hi_sm[g]
        rowid = m * bm + lax.broadcasted_iota(jnp.int32, (bm, 1), 0)
        mask = (rowid >= lo) & (rowid < hi)
        contrib = jnp.where(mask, out * wgt_ref[...], 0.0)
        prev_m = m_sm[jnp.maximum(g - 1, 0)]
        first = (g == 0) | (m != prev_m)

        @pl.when(first)
        def _():
            y_ref[...] = contrib

        @pl.when(jnp.logical_not(first))
        def _():
            y_ref[...] += contrib

    return body


def _shared_up_body(x_ref, ws1_ref, ws2_ref, h_ref):
    xb = x_ref[...].astype(jnp.bfloat16)
    ws1 = ws1_ref[0].astype(jnp.bfloat16)
    ws2 = ws2_ref[0].astype(jnp.bfloat16)
    h1 = lax.dot_general(xb, ws1, (((1,), (1,)), ((), ())),
                         preferred_element_type=jnp.float32)
    h2 = lax.dot_general(xb, ws2, (((1,), (1,)), ((), ())),
                         preferred_element_type=jnp.float32)
    h_ref[0] = (h1 * lax.logistic(h1) * h2).astype(jnp.bfloat16)


def _shared_down_body(h0_ref, h1_ref, h2_ref, h3_ref, wsp_ref, y_ref):
    hcat = jnp.concatenate(
        [h0_ref[0], h1_ref[0], h2_ref[0], h3_ref[0]], axis=1)
    wsp = wsp_ref[...].astype(jnp.bfloat16)
    y_ref[...] = lax.dot_general(hcat, wsp, (((1,), (1,)), ((), ())),
                                 preferred_element_type=jnp.float32)


def _final_add_body(ys_ref, g0_ref, g1_ref, y_ref):
    y_ref[...] = ys_ref[...] + g0_ref[...] + g1_ref[...]


def _make_sc_gather(nc, nw, rows_per_w, ch):
    def body(x_hbm, tok_hbm, out_hbm, idx_v, rows_v, sem):
        wid = lax.axis_index("s") * nc + lax.axis_index("c")
        base = wid * rows_per_w

        def chunk(c, carry):
            off = base + c * ch
            pltpu.sync_copy(tok_hbm.at[pl.ds(off, ch)], idx_v)
            pltpu.async_copy(x_hbm.at[idx_v], rows_v, sem).wait()
            pltpu.sync_copy(rows_v, out_hbm.at[pl.ds(off, ch)])
            return carry

        lax.fori_loop(0, rows_per_w // ch, chunk, 0)

    return body


def kernel(x, gate_W, W1, W2, Wp, Ws1, Ws2, Wsp):
    Bx, Tx, Cx = x.shape
    T = Bx * Tx
    E, INTER, C = W1.shape
    INTER_S = Ws1.shape[0]
    K = 2
    A = T * K
    x_flat = x.reshape(T, C)

    topk_idx, topk_w = pl.pallas_call(
        _router_body,
        out_shape=(jax.ShapeDtypeStruct((T, K), jnp.int32),
                   jax.ShapeDtypeStruct((T, K), jnp.float32)),
    )(x_flat, gate_W)

    BM = 256
    MT = A // BM
    G = MT + E - 1
    e_flat = topk_idx.reshape(A)
    w_flat = topk_w.reshape(A)
    sort_idx = jnp.argsort(e_flat, stable=True).astype(jnp.int32)
    tok_sorted = (sort_idx // K).astype(jnp.int32)
    w_sorted = w_flat[sort_idx].reshape(A, 1)
    inv = jnp.zeros((A,), jnp.int32).at[sort_idx].set(
        jnp.arange(A, dtype=jnp.int32))
    p0 = inv[0::K]
    p1 = inv[1::K]
    sizes = jnp.bincount(e_flat, length=E).astype(jnp.int32)
    offs = jnp.concatenate([jnp.zeros((1,), jnp.int32),
                            jnp.cumsum(sizes).astype(jnp.int32)])
    t_lo = offs[:-1] // BM
    t_hi = (offs[1:] - 1) // BM
    n_e = jnp.where(sizes > 0, t_hi - t_lo + 1, 0).astype(jnp.int32)
    cs = jnp.cumsum(n_e).astype(jnp.int32)
    garange = jnp.arange(G, dtype=jnp.int32)
    e_g = jnp.searchsorted(cs, garange, side='right').astype(jnp.int32)
    valid = e_g < E
    e_g = jnp.minimum(e_g, E - 1)
    start_e = cs[e_g] - n_e[e_g]
    m_g = jnp.where(valid, t_lo[e_g] + garange - start_e, MT - 1)
    lo_g = jnp.where(valid, jnp.maximum(m_g * BM, offs[:-1][e_g]), 0)
    hi_g = jnp.where(valid, jnp.minimum((m_g + 1) * BM, offs[1:][e_g]), 0)

    NIS = 4
    BIS = INTER_S // NIS
    BMS = min(256, T)
    tb = T // BMS
    Ws1r = Ws1.reshape(NIS, BIS, C)
    Ws2r = Ws2.reshape(NIS, BIS, C)
    Hs = pl.pallas_call(
        _shared_up_body,
        grid=(NIS, tb),
        in_specs=[
            pl.BlockSpec((BMS, C), lambda i, t: (t, 0)),
            pl.BlockSpec((1, BIS, C), lambda i, t: (i, 0, 0)),
            pl.BlockSpec((1, BIS, C), lambda i, t: (i, 0, 0)),
        ],
        out_specs=pl.BlockSpec((1, BMS, BIS), lambda i, t: (i, t, 0)),
        out_shape=jax.ShapeDtypeStruct((NIS, T, BIS), jnp.bfloat16),
        compiler_params=pltpu.CompilerParams(
            dimension_semantics=("arbitrary", "arbitrary")),
    )(x_flat, Ws1r, Ws2r)

    info = plsc.get_sparse_core_info()
    NC, NS = info.num_cores, info.num_subcores
    NW = NC * NS
    RPW = A // NW
    CH = 32
    mesh = plsc.VectorSubcoreMesh(core_axis_name="c", subcore_axis_name="s")
    xg = pl.kernel(
        _make_sc_gather(NC, NW, RPW, CH),
        out_type=jax.ShapeDtypeStruct((A, C), jnp.float32),
        mesh=mesh,
        scratch_types=[
            pltpu.VMEM((CH,), jnp.int32),
            pltpu.VMEM((CH, C), jnp.float32),
            pltpu.SemaphoreType.DMA,
        ],
    )(x_flat, tok_sorted)

    NI = 2
    BI = INTER // NI
    W1r = W1.reshape(E, NI, BI, C)
    W2r = W2.reshape(E, NI, BI, C)
    H = pl.pallas_call(
        _make_up_body(BM),
        grid_spec=pltpu.PrefetchScalarGridSpec(
            num_scalar_prefetch=4,
            grid=(NI, G),
            in_specs=[
                pl.BlockSpec((BM, C), lambda i, g, e, m, lo, hi: (m[g], 0)),
                pl.BlockSpec((1, 1, BI, C),
                             lambda i, g, e, m, lo, hi: (e[g], i, 0, 0)),
                pl.BlockSpec((1, 1, BI, C),
                             lambda i, g, e, m, lo, hi: (e[g], i, 0, 0)),
            ],
            out_specs=pl.BlockSpec((1, BM, BI),
                                   lambda i, g, e, m, lo, hi: (i, m[g], 0)),
        ),
        out_shape=jax.ShapeDtypeStruct((NI, A, BI), jnp.bfloat16),
        compiler_params=pltpu.CompilerParams(
            dimension_semantics=("arbitrary", "arbitrary")),
    )(e_g, m_g, lo_g, hi_g, xg, W1r, W2r)

    yg = pl.pallas_call(
        _make_down_body(BM),
        grid_spec=pltpu.PrefetchScalarGridSpec(
            num_scalar_prefetch=4,
            grid=(G,),
            in_specs=[
                pl.BlockSpec((1, BM, BI), lambda g, e, m, lo, hi: (0, m[g], 0)),
                pl.BlockSpec((1, BM, BI), lambda g, e, m, lo, hi: (1, m[g], 0)),
                pl.BlockSpec((BM, 1), lambda g, e, m, lo, hi: (m[g], 0)),
                pl.BlockSpec((1, C, INTER),
                             lambda g, e, m, lo, hi: (e[g], 0, 0)),
            ],
            out_specs=pl.BlockSpec((BM, C), lambda g, e, m, lo, hi: (m[g], 0)),
        ),
        out_shape=jax.ShapeDtypeStruct((A, C), jnp.float32),
        compiler_params=pltpu.CompilerParams(
            dimension_semantics=("arbitrary",)),
    )(e_g, m_g, lo_g, hi_g, H, H, w_sorted, Wp)

    pcat = jnp.concatenate([p0, p1])
    g01 = pl.kernel(
        _make_sc_gather(NC, NW, RPW, CH),
        out_type=jax.ShapeDtypeStruct((A, C), jnp.float32),
        mesh=mesh,
        scratch_types=[
            pltpu.VMEM((CH,), jnp.int32),
            pltpu.VMEM((CH, C), jnp.float32),
            pltpu.SemaphoreType.DMA,
        ],
    )(yg, pcat)

    ys = pl.pallas_call(
        _shared_down_body,
        grid=(tb,),
        in_specs=[
            pl.BlockSpec((1, BMS, BIS), lambda t: (0, t, 0)),
            pl.BlockSpec((1, BMS, BIS), lambda t: (1, t, 0)),
            pl.BlockSpec((1, BMS, BIS), lambda t: (2, t, 0)),
            pl.BlockSpec((1, BMS, BIS), lambda t: (3, t, 0)),
            pl.BlockSpec((C, INTER_S), lambda t: (0, 0),
                         pipeline_mode=pl.Buffered(buffer_count=1)),
        ],
        out_specs=pl.BlockSpec((BMS, C), lambda t: (t, 0)),
        out_shape=jax.ShapeDtypeStruct((T, C), jnp.float32),
        compiler_params=pltpu.CompilerParams(
            dimension_semantics=("arbitrary",)),
    )(Hs, Hs, Hs, Hs, Wsp)

    y = pl.pallas_call(
        _final_add_body,
        grid=(tb,),
        in_specs=[
            pl.BlockSpec((BMS, C), lambda t: (t, 0)),
            pl.BlockSpec((BMS, C), lambda t: (t, 0)),
            pl.BlockSpec((BMS, C), lambda t, tbk=tb: (t + tbk, 0)),
        ],
        out_specs=pl.BlockSpec((BMS, C), lambda t: (t, 0)),
        out_shape=jax.ShapeDtypeStruct((T, C), jnp.float32),
        compiler_params=pltpu.CompilerParams(
            dimension_semantics=("arbitrary",)),
    )(ys, g01, g01)

    return y.reshape(Bx, Tx, Cx)


# speedup vs baseline: 1.0468x; 1.0468x over previous
import functools

import jax
import jax.numpy as jnp
from jax import lax
from jax.experimental import pallas as pl
from jax.experimental.pallas import tpu as pltpu
from jax.experimental.pallas import tpu_sc as plsc


def _router_body(x_ref, gw_ref, idx_ref, w_ref):
    x = x_ref[...]
    gw = gw_ref[...]
    logits = lax.dot_general(x, gw, (((1,), (1,)), ((), ())),
                             preferred_element_type=jnp.float32)
    m = jnp.max(logits, axis=1, keepdims=True)
    ex = jnp.exp(logits - m)
    probs = ex / jnp.sum(ex, axis=1, keepdims=True)
    col = lax.broadcasted_iota(jnp.int32, probs.shape, 1)
    big = jnp.int32(2 ** 30)
    p1 = jnp.max(probs, axis=1, keepdims=True)
    i1 = jnp.min(jnp.where(probs == p1, col, big), axis=1, keepdims=True)
    mask1 = col == i1
    probs2 = jnp.where(mask1, -1.0, probs)
    p2 = jnp.max(probs2, axis=1, keepdims=True)
    i2 = jnp.min(jnp.where(probs2 == p2, col, big), axis=1, keepdims=True)
    idx_ref[...] = jnp.concatenate([i1, i2], axis=1)
    w_ref[...] = jnp.concatenate([p1, p2], axis=1)


def _swiglu(x, w1, w2):
    h1 = lax.dot_general(x, w1, (((1,), (1,)), ((), ())),
                         preferred_element_type=jnp.float32)
    h2 = lax.dot_general(x, w2, (((1,), (1,)), ((), ())),
                         preferred_element_type=jnp.float32)
    return (h1 * lax.logistic(h1) * h2).astype(jnp.bfloat16)


def _make_up_body(bm):
    def body(e_sm, m_sm, lo_sm, hi_sm, xg_ref, w1_ref, w2_ref, h_ref):
        g = pl.program_id(1)
        xv = xg_ref[...].astype(jnp.bfloat16)
        w1 = w1_ref[0, 0].astype(jnp.bfloat16)
        w2 = w2_ref[0, 0].astype(jnp.bfloat16)
        h1 = lax.dot_general(xv, w1, (((1,), (1,)), ((), ())),
                             preferred_element_type=jnp.float32)
        h2 = lax.dot_general(xv, w2, (((1,), (1,)), ((), ())),
                             preferred_element_type=jnp.float32)
        hv = (h1 * lax.logistic(h1) * h2).astype(jnp.bfloat16)
        m = m_sm[g]
        lo = lo_sm[g]
        hi = hi_sm[g]
        rowid = m * bm + lax.broadcasted_iota(jnp.int32, (bm, 1), 0)
        mask = (rowid >= lo) & (rowid < hi)
        h_ref[0] = jnp.where(mask, hv, h_ref[0])

    return body


def _make_down_body(bm):
    def body(e_sm, m_sm, lo_sm, hi_sm, h0_ref, h1_ref, wgt_ref, wp_ref,
             y_ref):
        g = pl.program_id(0)
        hcat = jnp.concatenate([h0_ref[0], h1_ref[0]], axis=1)
        wp = wp_ref[0].astype(jnp.bfloat16)
        out = lax.dot_general(hcat, wp, (((1,), (1,)), ((), ())),
                              preferred_element_type=jnp.float32)
        m = m_sm[g]
        lo = lo_sm[g]
        hi = hi_sm[g]
        rowid = m * bm + lax.broadcasted_iota(jnp.int32, (bm, 1), 0)
        mask = (rowid >= lo) & (rowid < hi)
        contrib = jnp.where(mask, out * wgt_ref[...], 0.0)
        prev_m = m_sm[jnp.maximum(g - 1, 0)]
        first = (g == 0) | (m != prev_m)

        @pl.when(first)
        def _():
            y_ref[...] = contrib

        @pl.when(jnp.logical_not(first))
        def _():
            y_ref[...] += contrib

    return body


def _shared_up_body(x_ref, ws1_ref, ws2_ref, h_ref):
    xb = x_ref[...].astype(jnp.bfloat16)
    ws1 = ws1_ref[0].astype(jnp.bfloat16)
    ws2 = ws2_ref[0].astype(jnp.bfloat16)
    h1 = lax.dot_general(xb, ws1, (((1,), (1,)), ((), ())),
                         preferred_element_type=jnp.float32)
    h2 = lax.dot_general(xb, ws2, (((1,), (1,)), ((), ())),
                         preferred_element_type=jnp.float32)
    h_ref[0] = (h1 * lax.logistic(h1) * h2).astype(jnp.bfloat16)


def _shared_down_body(h0_ref, h1_ref, h2_ref, h3_ref, wsp_ref, g0_ref,
                      g1_ref, y_ref):
    hcat = jnp.concatenate(
        [h0_ref[0], h1_ref[0], h2_ref[0], h3_ref[0]], axis=1)
    wsp = wsp_ref[...].astype(jnp.bfloat16)
    out = lax.dot_general(hcat, wsp, (((1,), (1,)), ((), ())),
                          preferred_element_type=jnp.float32)
    y_ref[...] = out + g0_ref[...] + g1_ref[...]


def _make_sc_gather(nc, nw, rows_per_w, ch):
    def body(x_hbm, tok_hbm, out_hbm, idx_v, rows_v, sem):
        wid = lax.axis_index("s") * nc + lax.axis_index("c")
        base = wid * rows_per_w

        def chunk(c, carry):
            off = base + c * ch
            pltpu.sync_copy(tok_hbm.at[pl.ds(off, ch)], idx_v)
            pltpu.async_copy(x_hbm.at[idx_v], rows_v, sem).wait()
            pltpu.sync_copy(rows_v, out_hbm.at[pl.ds(off, ch)])
            return carry

        lax.fori_loop(0, rows_per_w // ch, chunk, 0)

    return body


def kernel(x, gate_W, W1, W2, Wp, Ws1, Ws2, Wsp):
    Bx, Tx, Cx = x.shape
    T = Bx * Tx
    E, INTER, C = W1.shape
    INTER_S = Ws1.shape[0]
    K = 2
    A = T * K
    x_flat = x.reshape(T, C)

    topk_idx, topk_w = pl.pallas_call(
        _router_body,
        out_shape=(jax.ShapeDtypeStruct((T, K), jnp.int32),
                   jax.ShapeDtypeStruct((T, K), jnp.float32)),
    )(x_flat, gate_W)

    BM = 256
    MT = A // BM
    G = MT + E - 1
    e_flat = topk_idx.reshape(A)
    w_flat = topk_w.reshape(A)
    sort_idx = jnp.argsort(e_flat, stable=True).astype(jnp.int32)
    tok_sorted = (sort_idx // K).astype(jnp.int32)
    w_sorted = w_flat[sort_idx].reshape(A, 1)
    inv = jnp.zeros((A,), jnp.int32).at[sort_idx].set(
        jnp.arange(A, dtype=jnp.int32))
    p0 = inv[0::K]
    p1 = inv[1::K]
    sizes = jnp.bincount(e_flat, length=E).astype(jnp.int32)
    offs = jnp.concatenate([jnp.zeros((1,), jnp.int32),
                            jnp.cumsum(sizes).astype(jnp.int32)])
    t_lo = offs[:-1] // BM
    t_hi = (offs[1:] - 1) // BM
    n_e = jnp.where(sizes > 0, t_hi - t_lo + 1, 0).astype(jnp.int32)
    cs = jnp.cumsum(n_e).astype(jnp.int32)
    garange = jnp.arange(G, dtype=jnp.int32)
    e_g = jnp.searchsorted(cs, garange, side='right').astype(jnp.int32)
    valid = e_g < E
    e_g = jnp.minimum(e_g, E - 1)
    start_e = cs[e_g] - n_e[e_g]
    m_g = jnp.where(valid, t_lo[e_g] + garange - start_e, MT - 1)
    lo_g = jnp.where(valid, jnp.maximum(m_g * BM, offs[:-1][e_g]), 0)
    hi_g = jnp.where(valid, jnp.minimum((m_g + 1) * BM, offs[1:][e_g]), 0)

    info = plsc.get_sparse_core_info()
    NC, NS = info.num_cores, info.num_subcores
    NW = NC * NS
    RPW = A // NW
    CH = 32
    mesh = plsc.VectorSubcoreMesh(core_axis_name="c", subcore_axis_name="s")
    xg = pl.kernel(
        _make_sc_gather(NC, NW, RPW, CH),
        out_type=jax.ShapeDtypeStruct((A, C), jnp.float32),
        mesh=mesh,
        scratch_types=[
            pltpu.VMEM((CH,), jnp.int32),
            pltpu.VMEM((CH, C), jnp.float32),
            pltpu.SemaphoreType.DMA,
        ],
    )(x_flat, tok_sorted)

    NI = 2
    BI = INTER // NI
    W1r = W1.reshape(E, NI, BI, C)
    W2r = W2.reshape(E, NI, BI, C)
    H = pl.pallas_call(
        _make_up_body(BM),
        grid_spec=pltpu.PrefetchScalarGridSpec(
            num_scalar_prefetch=4,
            grid=(NI, G),
            in_specs=[
                pl.BlockSpec((BM, C), lambda i, g, e, m, lo, hi: (m[g], 0)),
                pl.BlockSpec((1, 1, BI, C),
                             lambda i, g, e, m, lo, hi: (e[g], i, 0, 0)),
                pl.BlockSpec((1, 1, BI, C),
                             lambda i, g, e, m, lo, hi: (e[g], i, 0, 0)),
            ],
            out_specs=pl.BlockSpec((1, BM, BI),
                                   lambda i, g, e, m, lo, hi: (i, m[g], 0)),
        ),
        out_shape=jax.ShapeDtypeStruct((NI, A, BI), jnp.bfloat16),
        compiler_params=pltpu.CompilerParams(
            dimension_semantics=("arbitrary", "arbitrary")),
    )(e_g, m_g, lo_g, hi_g, xg, W1r, W2r)

    yg = pl.pallas_call(
        _make_down_body(BM),
        grid_spec=pltpu.PrefetchScalarGridSpec(
            num_scalar_prefetch=4,
            grid=(G,),
            in_specs=[
                pl.BlockSpec((1, BM, BI), lambda g, e, m, lo, hi: (0, m[g], 0)),
                pl.BlockSpec((1, BM, BI), lambda g, e, m, lo, hi: (1, m[g], 0)),
                pl.BlockSpec((BM, 1), lambda g, e, m, lo, hi: (m[g], 0)),
                pl.BlockSpec((1, C, INTER),
                             lambda g, e, m, lo, hi: (e[g], 0, 0)),
            ],
            out_specs=pl.BlockSpec((BM, C), lambda g, e, m, lo, hi: (m[g], 0)),
        ),
        out_shape=jax.ShapeDtypeStruct((A, C), jnp.float32),
        compiler_params=pltpu.CompilerParams(
            dimension_semantics=("arbitrary",)),
    )(e_g, m_g, lo_g, hi_g, H, H, w_sorted, Wp)

    pcat = jnp.concatenate([p0, p1])
    g01 = pl.kernel(
        _make_sc_gather(NC, NW, RPW, CH),
        out_type=jax.ShapeDtypeStruct((A, C), jnp.float32),
        mesh=mesh,
        scratch_types=[
            pltpu.VMEM((CH,), jnp.int32),
            pltpu.VMEM((CH, C), jnp.float32),
            pltpu.SemaphoreType.DMA,
        ],
    )(yg, pcat)

    NIS = 4
    BIS = INTER_S // NIS
    BMS = min(256, T)
    tb = T // BMS
    Ws1r = Ws1.reshape(NIS, BIS, C)
    Ws2r = Ws2.reshape(NIS, BIS, C)
    Hs = pl.pallas_call(
        _shared_up_body,
        grid=(NIS, tb),
        in_specs=[
            pl.BlockSpec((BMS, C), lambda i, t: (t, 0)),
            pl.BlockSpec((1, BIS, C), lambda i, t: (i, 0, 0)),
            pl.BlockSpec((1, BIS, C), lambda i, t: (i, 0, 0)),
        ],
        out_specs=pl.BlockSpec((1, BMS, BIS), lambda i, t: (i, t, 0)),
        out_shape=jax.ShapeDtypeStruct((NIS, T, BIS), jnp.bfloat16),
        compiler_params=pltpu.CompilerParams(
            dimension_semantics=("arbitrary", "arbitrary")),
    )(x_flat, Ws1r, Ws2r)

    y = pl.pallas_call(
        _shared_down_body,
        grid=(tb,),
        in_specs=[
            pl.BlockSpec((1, BMS, BIS), lambda t: (0, t, 0)),
            pl.BlockSpec((1, BMS, BIS), lambda t: (1, t, 0)),
            pl.BlockSpec((1, BMS, BIS), lambda t: (2, t, 0)),
            pl.BlockSpec((1, BMS, BIS), lambda t: (3, t, 0)),
            pl.BlockSpec((C, INTER_S), lambda t: (0, 0),
                         pipeline_mode=pl.Buffered(buffer_count=1)),
            pl.BlockSpec((BMS, C), lambda t: (t, 0)),
            pl.BlockSpec((BMS, C), lambda t, tbk=tb: (t + tbk, 0)),
        ],
        out_specs=pl.BlockSpec((BMS, C), lambda t: (t, 0)),
        out_shape=jax.ShapeDtypeStruct((T, C), jnp.float32),
        compiler_params=pltpu.CompilerParams(
            dimension_semantics=("arbitrary",)),
    )(Hs, Hs, Hs, Hs, Wsp, g01, g01)

    return y.reshape(Bx, Tx, Cx)
